# Initial kernel scaffold; baseline (speedup 1.0000x reference)
import jax
import jax.numpy as jnp
from jax.experimental import pallas as pl


def kernel(features, labels, neighborhood_ids, closest_neighbors, unit_search_neighbors, neighborhood_explore_units, unit_means):
    raise NotImplementedError("write your pallas kernel here")


# speedup vs baseline: 14.3749x; 14.3749x over previous
import functools

import jax
import jax.numpy as jnp
from jax import lax
from jax.experimental import pallas as pl
from jax.experimental.pallas import tpu as pltpu
from jax.experimental.pallas import tpu_sc as plsc

N_SPIKES_ = 131072
N_UNITS_ = 512
N_NEIGHB_ = 64
D_ = 32
L = 16
NC, NS = 2, 16
NW = NC * NS
SPW = N_SPIKES_ // NW
CH = 256
NCH = SPW // CH
NG = CH // L


def _sc_body(fT, lab_h, nb_h, cl_h, usn_h, expl_h, mu_h,
             ntop_h, q_h, nstat_h, m_h, cnt_h, elbo_h,
             mu_v, cl_v, usn_v, expl_v, nexp_v, nstat_v, m_v, cnt_v,
             fT_v, lab_v, nb_v, ntop_v, q_v, elbo_v, sem):
    del sem
    wid = lax.axis_index("s") * NC + lax.axis_index("c")

    pltpu.sync_copy(mu_h, mu_v)
    pltpu.sync_copy(cl_h, cl_v)
    pltpu.sync_copy(usn_h, usn_v)
    pltpu.sync_copy(expl_h, expl_v)

    iota16 = lax.iota(jnp.int32, L)
    one_i = jnp.ones((L,), jnp.int32)
    zero_i = jnp.zeros((L,), jnp.int32)

    def nexp_body(c, accs):
        col = jnp.full((L,), 0, jnp.int32) + c
        new = []
        for rr in range(N_NEIGHB_ // L):
            rows = iota16 + rr * L
            v = plsc.load_gather(expl_v, [rows, col])
            new.append(accs[rr] + jnp.where(v < N_UNITS_, one_i, zero_i))
        return tuple(new)
    accs = lax.fori_loop(0, N_UNITS_, nexp_body,
                         tuple(zero_i for _ in range(N_NEIGHB_ // L)))
    for rr in range(N_NEIGHB_ // L):
        nexp_v[pl.ds(rr * L, L)] = accs[rr]

    zf = jnp.zeros((L,), jnp.float32)

    def zero_m(i, carry):
        for k in range(8):
            m_v[pl.ds(i * 8 * L + k * L, L)] = zf
        return carry
    lax.fori_loop(0, (N_UNITS_ * D_) // (8 * L), zero_m, None)

    def zero_cnt(i, carry):
        for k in range(8):
            cnt_v[pl.ds(i * 8 * L + k * L, L)] = zf
        return carry
    lax.fori_loop(0, (N_UNITS_ * N_NEIGHB_) // (8 * L), zero_cnt, None)

    for i in range(N_UNITS_ // L):
        nstat_v[pl.ds(i * L, L)] = zf
    elbo_v[...] = zf

    def chunk_body(ci, carry):
        base = wid * SPW + ci * CH
        pltpu.sync_copy(fT.at[:, pl.ds(base, CH)], fT_v)
        pltpu.sync_copy(lab_h.at[pl.ds(base, CH)], lab_v)
        pltpu.sync_copy(nb_h.at[pl.ds(base, CH)], nb_v)

        def group_body(g, carry2):
            gb = g * L
            lab = lab_v[pl.ds(gb, L)]
            nbv = nb_v[pl.ds(gb, L)]
            iota = lax.iota(jnp.int32, L)
            sidx = base + gb + iota

            cands = []
            for j in range(3):
                cands.append(plsc.load_gather(
                    cl_v, [lab, jnp.full((L,), j, jnp.int32)]))
            for j in range(3):
                for k in range(2):
                    cands.append(plsc.load_gather(
                        usn_v, [cands[j], jnp.full((L,), k, jnp.int32)]))
            ne = plsc.load_gather(nexp_v, [nbv])
            ne = jnp.maximum(ne, 1)
            q0 = (sidx.astype(jnp.float32) / ne.astype(jnp.float32)
                  ).astype(jnp.int32)
            targ = sidx - q0 * ne
            targ = jnp.where(targ < 0, targ + ne, targ)
            targ = jnp.where(targ >= ne, targ - ne, targ)
            cands.append(plsc.load_gather(expl_v, [nbv, targ]))

            lls = []
            for j in range(10):
                cj = cands[j] * D_
                prods = []
                for d in range(D_):
                    md = plsc.load_gather(mu_v, [cj + d])
                    fd = fT_v[d, pl.ds(gb, L)]
                    t = fd - md
                    prods.append(t * t)
                while len(prods) > 1:
                    prods = [prods[i] + prods[i + 1]
                             for i in range(0, len(prods), 2)]
                lls.append(-0.5 * prods[0])

            neg_inf = jnp.full((L,), -jnp.inf, jnp.float32)
            chosen_pos, vals_sel, cand_sel = [], [], []
            for r in range(3):
                best, bp = neg_inf, jnp.zeros((L,), jnp.int32)
                bc = jnp.zeros((L,), jnp.int32)
                for j in range(10):
                    ok = lls[j] > best
                    for p in chosen_pos:
                        ok = ok & (p != j)
                    best = jnp.where(ok, lls[j], best)
                    bp = jnp.where(ok, jnp.full((L,), j, jnp.int32), bp)
                    bc = jnp.where(ok, cands[j], bc)
                chosen_pos.append(bp)
                vals_sel.append(best)
                cand_sel.append(bc)

            mx = jnp.maximum(vals_sel[0], jnp.maximum(vals_sel[1],
                                                      vals_sel[2]))
            es = [jnp.exp(v - mx) for v in vals_sel]
            ssum = (es[0] + es[1]) + es[2]
            qs = [e / ssum for e in es]

            for r in range(3):
                ntop_v[r, pl.ds(gb, L)] = cand_sel[r]
                q_v[r, pl.ds(gb, L)] = qs[r]

            contrib = (qs[0] * vals_sel[0] + qs[1] * vals_sel[1]
                       + qs[2] * vals_sel[2])
            elbo_v[...] = elbo_v[...] + contrib

            for r in range(3):
                plsc.addupdate_scatter(nstat_v, [cand_sel[r]], qs[r])
            for r in range(3):
                c32 = cand_sel[r] * D_
                for d in range(D_):
                    fd = fT_v[d, pl.ds(gb, L)]
                    plsc.addupdate_scatter(m_v, [c32 + d], qs[r] * fd)
            onesf = jnp.ones((L,), jnp.float32)
            for j in range(10):
                plsc.addupdate_scatter(
                    cnt_v, [cands[j] * N_NEIGHB_ + nbv], onesf)
            return carry2
        lax.fori_loop(0, NG, group_body, None)

        pltpu.sync_copy(ntop_v, ntop_h.at[:, pl.ds(base, CH)])
        pltpu.sync_copy(q_v, q_h.at[:, pl.ds(base, CH)])
        return carry
    lax.fori_loop(0, NCH, chunk_body, None)

    pltpu.sync_copy(nstat_v, nstat_h.at[wid])
    pltpu.sync_copy(m_v, m_h.at[wid])
    pltpu.sync_copy(cnt_v, cnt_h.at[wid])
    pltpu.sync_copy(elbo_v, elbo_h.at[wid])


def kernel(features, labels, neighborhood_ids, closest_neighbors,
           unit_search_neighbors, neighborhood_explore_units, unit_means):
    n_spikes, d = features.shape
    n_units = unit_means.shape[0]
    assert (n_spikes, d, n_units) == (N_SPIKES_, D_, N_UNITS_)

    fT = features.T
    mu_flat = unit_means.reshape(-1)

    mesh = plsc.VectorSubcoreMesh(core_axis_name="c", subcore_axis_name="s")
    f32, i32 = jnp.float32, jnp.int32
    out_type = (
        jax.ShapeDtypeStruct((3, N_SPIKES_), i32),
        jax.ShapeDtypeStruct((3, N_SPIKES_), f32),
        jax.ShapeDtypeStruct((NW, N_UNITS_), f32),
        jax.ShapeDtypeStruct((NW, N_UNITS_ * D_), f32),
        jax.ShapeDtypeStruct((NW, N_UNITS_ * N_NEIGHB_), f32),
        jax.ShapeDtypeStruct((NW, L), f32),
    )
    scratch_types = [
        pltpu.VMEM((N_UNITS_ * D_,), f32),
        pltpu.VMEM((N_UNITS_, 3), i32),
        pltpu.VMEM((N_UNITS_, 2), i32),
        pltpu.VMEM((N_NEIGHB_, N_UNITS_), i32),
        pltpu.VMEM((N_NEIGHB_,), i32),
        pltpu.VMEM((N_UNITS_,), f32),
        pltpu.VMEM((N_UNITS_ * D_,), f32),
        pltpu.VMEM((N_UNITS_ * N_NEIGHB_,), f32),
        pltpu.VMEM((D_, CH), f32),
        pltpu.VMEM((CH,), i32),
        pltpu.VMEM((CH,), i32),
        pltpu.VMEM((3, CH), i32),
        pltpu.VMEM((3, CH), f32),
        pltpu.VMEM((L,), f32),
        pltpu.SemaphoreType.DMA,
    ]
    run = pl.kernel(_sc_body, out_type=out_type, mesh=mesh,
                    scratch_types=scratch_types,
                    compiler_params=pltpu.CompilerParams(
                        use_tc_tiling_on_sc=False,
                        needs_layout_passes=False))
    ntop_t, q_t, nstat_p, m_p, cnt_p, elbo_p = run(
        fT, labels, neighborhood_ids, closest_neighbors,
        unit_search_neighbors, neighborhood_explore_units, mu_flat)

    new_top = ntop_t.T
    Q = q_t.T
    Nstat = nstat_p.sum(axis=0)
    m = m_p.sum(axis=0).reshape(N_UNITS_, D_)
    counts = cnt_p.sum(axis=0).reshape(N_UNITS_, N_NEIGHB_)
    obs_elbo = elbo_p.sum()
    return new_top, Q, Nstat, m, counts, obs_elbo


# speedup vs baseline: 15.9599x; 1.1103x over previous
import functools

import jax
import jax.numpy as jnp
from jax import lax
from jax.experimental import pallas as pl
from jax.experimental.pallas import tpu as pltpu
from jax.experimental.pallas import tpu_sc as plsc

N_SPIKES_ = 131072
N_UNITS_ = 512
N_NEIGHB_ = 64
D_ = 32
L = 16
NC, NS = 2, 16
NW = NC * NS
SPW = N_SPIKES_ // NW
CH = 256
NCH = SPW // CH
NG = CH // L


def _sc_body(fT, lab_h, nb_h, cl_h, usn_h, expl_h, mu_h,
             ntop_h, q_h, nstat_h, m_h, cnt_h, elbo_h,
             mu_v, cl_v, usn_v, expl_v, nexp_v, nstat_v, m_v, cnt_v,
             fT_v, lab_v, nb_v, ntop_v, q_v, elbo_v, sem):
    del sem
    wid = lax.axis_index("s") * NC + lax.axis_index("c")

    pltpu.sync_copy(mu_h, mu_v)
    pltpu.sync_copy(cl_h, cl_v)
    pltpu.sync_copy(usn_h, usn_v)
    pltpu.sync_copy(expl_h, expl_v)

    iota16 = lax.iota(jnp.int32, L)
    one_i = jnp.ones((L,), jnp.int32)
    zero_i = jnp.zeros((L,), jnp.int32)

    def nexp_body(c, accs):
        col = jnp.full((L,), 0, jnp.int32) + c
        new = []
        for rr in range(N_NEIGHB_ // L):
            rows = iota16 + rr * L
            v = plsc.load_gather(expl_v, [rows, col])
            new.append(accs[rr] + jnp.where(v < N_UNITS_, one_i, zero_i))
        return tuple(new)
    accs = lax.fori_loop(0, N_UNITS_, nexp_body,
                         tuple(zero_i for _ in range(N_NEIGHB_ // L)))
    for rr in range(N_NEIGHB_ // L):
        nexp_v[pl.ds(rr * L, L)] = accs[rr]

    zf = jnp.zeros((L,), jnp.float32)

    def zero_m(i, carry):
        for k in range(8):
            m_v[pl.ds(i * 8 * L + k * L, L)] = zf
        return carry
    lax.fori_loop(0, (N_UNITS_ * D_) // (8 * L), zero_m, None)

    def zero_cnt(i, carry):
        for k in range(8):
            cnt_v[pl.ds(i * 8 * L + k * L, L)] = zf
        return carry
    lax.fori_loop(0, (N_UNITS_ * N_NEIGHB_) // (8 * L), zero_cnt, None)

    for i in range(N_UNITS_ // L):
        nstat_v[pl.ds(i * L, L)] = zf
    elbo_v[...] = zf

    def chunk_body(ci, elbo_c):
        base = wid * SPW + ci * CH
        pltpu.sync_copy(fT.at[:, pl.ds(base, CH)], fT_v)
        pltpu.sync_copy(lab_h.at[pl.ds(base, CH)], lab_v)
        pltpu.sync_copy(nb_h.at[pl.ds(base, CH)], nb_v)

        def group_body(g, elbo_acc):
            gb = g * L
            lab = lab_v[pl.ds(gb, L)]
            nbv = nb_v[pl.ds(gb, L)]
            iota = lax.iota(jnp.int32, L)
            sidx = base + gb + iota
            fds = [fT_v[d, pl.ds(gb, L)] for d in range(D_)]

            cands = []
            for j in range(3):
                cands.append(plsc.load_gather(
                    cl_v, [lab, jnp.full((L,), j, jnp.int32)]))
            for j in range(3):
                for k in range(2):
                    cands.append(plsc.load_gather(
                        usn_v, [cands[j], jnp.full((L,), k, jnp.int32)]))
            ne = plsc.load_gather(nexp_v, [nbv])
            ne = jnp.maximum(ne, 1)
            q0 = (sidx.astype(jnp.float32) / ne.astype(jnp.float32)
                  ).astype(jnp.int32)
            targ = sidx - q0 * ne
            targ = jnp.where(targ < 0, targ + ne, targ)
            targ = jnp.where(targ >= ne, targ - ne, targ)
            cands.append(plsc.load_gather(expl_v, [nbv, targ]))

            lls = []
            for j in range(10):
                cj = cands[j] * D_
                prods = []
                for d in range(D_):
                    md = plsc.load_gather(mu_v, [cj + d])
                    t = fds[d] - md
                    prods.append(t * t)
                while len(prods) > 1:
                    prods = [prods[i] + prods[i + 1]
                             for i in range(0, len(prods), 2)]
                lls.append(-0.5 * prods[0])

            neg_inf = jnp.full((L,), -jnp.inf, jnp.float32)
            chosen_pos, vals_sel, cand_sel = [], [], []
            for r in range(3):
                best, bp = neg_inf, jnp.zeros((L,), jnp.int32)
                bc = jnp.zeros((L,), jnp.int32)
                for j in range(10):
                    ok = lls[j] > best
                    for p in chosen_pos:
                        ok = ok & (p != j)
                    best = jnp.where(ok, lls[j], best)
                    bp = jnp.where(ok, jnp.full((L,), j, jnp.int32), bp)
                    bc = jnp.where(ok, cands[j], bc)
                chosen_pos.append(bp)
                vals_sel.append(best)
                cand_sel.append(bc)

            mx = jnp.maximum(vals_sel[0], jnp.maximum(vals_sel[1],
                                                      vals_sel[2]))
            es = [jnp.exp(v - mx) for v in vals_sel]
            ssum = (es[0] + es[1]) + es[2]
            qs = [e / ssum for e in es]

            for r in range(3):
                ntop_v[r, pl.ds(gb, L)] = cand_sel[r]
                q_v[r, pl.ds(gb, L)] = qs[r]

            contrib = (qs[0] * vals_sel[0] + qs[1] * vals_sel[1]
                       + qs[2] * vals_sel[2])

            for r in range(3):
                plsc.addupdate_scatter(nstat_v, [cand_sel[r]], qs[r])
            for r in range(3):
                c32 = cand_sel[r] * D_
                for d in range(D_):
                    plsc.addupdate_scatter(m_v, [c32 + d], qs[r] * fds[d])
            onesf = jnp.ones((L,), jnp.float32)
            for j in range(10):
                plsc.addupdate_scatter(
                    cnt_v, [cands[j] * N_NEIGHB_ + nbv], onesf)
            return elbo_acc + contrib
        elbo_c = plsc.parallel_loop(0, NG, carry=elbo_c)(group_body)

        pltpu.sync_copy(ntop_v, ntop_h.at[:, pl.ds(base, CH)])
        pltpu.sync_copy(q_v, q_h.at[:, pl.ds(base, CH)])
        return elbo_c
    elbo_fin = lax.fori_loop(0, NCH, chunk_body, jnp.zeros((L,), jnp.float32))
    elbo_v[...] = elbo_fin

    pltpu.sync_copy(nstat_v, nstat_h.at[wid])
    pltpu.sync_copy(m_v, m_h.at[wid])
    pltpu.sync_copy(cnt_v, cnt_h.at[wid])
    pltpu.sync_copy(elbo_v, elbo_h.at[wid])


def kernel(features, labels, neighborhood_ids, closest_neighbors,
           unit_search_neighbors, neighborhood_explore_units, unit_means):
    n_spikes, d = features.shape
    n_units = unit_means.shape[0]
    assert (n_spikes, d, n_units) == (N_SPIKES_, D_, N_UNITS_)

    fT = features.T
    mu_flat = unit_means.reshape(-1)

    mesh = plsc.VectorSubcoreMesh(core_axis_name="c", subcore_axis_name="s")
    f32, i32 = jnp.float32, jnp.int32
    out_type = (
        jax.ShapeDtypeStruct((3, N_SPIKES_), i32),
        jax.ShapeDtypeStruct((3, N_SPIKES_), f32),
        jax.ShapeDtypeStruct((NW, N_UNITS_), f32),
        jax.ShapeDtypeStruct((NW, N_UNITS_ * D_), f32),
        jax.ShapeDtypeStruct((NW, N_UNITS_ * N_NEIGHB_), f32),
        jax.ShapeDtypeStruct((NW, L), f32),
    )
    scratch_types = [
        pltpu.VMEM((N_UNITS_ * D_,), f32),
        pltpu.VMEM((N_UNITS_, 3), i32),
        pltpu.VMEM((N_UNITS_, 2), i32),
        pltpu.VMEM((N_NEIGHB_, N_UNITS_), i32),
        pltpu.VMEM((N_NEIGHB_,), i32),
        pltpu.VMEM((N_UNITS_,), f32),
        pltpu.VMEM((N_UNITS_ * D_,), f32),
        pltpu.VMEM((N_UNITS_ * N_NEIGHB_,), f32),
        pltpu.VMEM((D_, CH), f32),
        pltpu.VMEM((CH,), i32),
        pltpu.VMEM((CH,), i32),
        pltpu.VMEM((3, CH), i32),
        pltpu.VMEM((3, CH), f32),
        pltpu.VMEM((L,), f32),
        pltpu.SemaphoreType.DMA,
    ]
    run = pl.kernel(_sc_body, out_type=out_type, mesh=mesh,
                    scratch_types=scratch_types,
                    compiler_params=pltpu.CompilerParams(
                        use_tc_tiling_on_sc=False,
                        needs_layout_passes=False))
    ntop_t, q_t, nstat_p, m_p, cnt_p, elbo_p = run(
        fT, labels, neighborhood_ids, closest_neighbors,
        unit_search_neighbors, neighborhood_explore_units, mu_flat)

    new_top = ntop_t.T
    Q = q_t.T
    Nstat = nstat_p.sum(axis=0)
    m = m_p.sum(axis=0).reshape(N_UNITS_, D_)
    counts = cnt_p.sum(axis=0).reshape(N_UNITS_, N_NEIGHB_)
    obs_elbo = elbo_p.sum()
    return new_top, Q, Nstat, m, counts, obs_elbo


# speedup vs baseline: 57.0969x; 3.5775x over previous
import jax
import jax.numpy as jnp
from jax import lax
from jax.experimental import pallas as pl
from jax.experimental.pallas import tpu as pltpu
from jax.experimental.pallas import tpu_sc as plsc

N_SPIKES_ = 131072
N_UNITS_ = 512
N_NEIGHB_ = 64
D_ = 32
L = 16
NC, NS = 2, 16
NW = NC * NS
SPW = N_SPIKES_ // NW
CH = 512
NCH = SPW // CH
NG = CH // L
DP = D_ + 1


def _sc_body(fT, lab_h, nb_h, cl_h, usn_h, expl_h, mu_h,
             ntop_h, q_h, nstat_h, m_h, cnt_h, elbo_h,
             mu_v, cl_v, usn_v, expl_v, nexp_v, nstat_v, m_v, cnt_v,
             fT_v, lab_v, nb_v, ntop_v, q_v, elbo_v, sem):
    del sem
    wid = lax.axis_index("s") * NC + lax.axis_index("c")

    pltpu.sync_copy(mu_h, mu_v)
    pltpu.sync_copy(cl_h, cl_v)
    pltpu.sync_copy(usn_h, usn_v)
    pltpu.sync_copy(expl_h, expl_v)

    iota16 = lax.iota(jnp.int32, L)
    one_i = jnp.ones((L,), jnp.int32)
    zero_i = jnp.zeros((L,), jnp.int32)

    def nexp_body(c, accs):
        col = jnp.full((L,), 0, jnp.int32) + c
        new = []
        for rr in range(N_NEIGHB_ // L):
            rows = iota16 + rr * L
            v = plsc.load_gather(expl_v, [rows, col])
            new.append(accs[rr] + jnp.where(v < N_UNITS_, one_i, zero_i))
        return tuple(new)
    accs = lax.fori_loop(0, N_UNITS_, nexp_body,
                         tuple(zero_i for _ in range(N_NEIGHB_ // L)))
    for rr in range(N_NEIGHB_ // L):
        nexp_v[pl.ds(rr * L, L)] = accs[rr]

    zf = jnp.zeros((L,), jnp.float32)

    def zero_m(i, carry):
        for k in range(8):
            m_v[pl.ds(i * 8 * L + k * L, L)] = zf
        return carry
    lax.fori_loop(0, (N_UNITS_ * DP) // (8 * L), zero_m, None)

    def zero_cnt(i, carry):
        for k in range(8):
            cnt_v[pl.ds(i * 8 * L + k * L, L)] = zf
        return carry
    lax.fori_loop(0, (N_UNITS_ * N_NEIGHB_) // (8 * L), zero_cnt, None)

    for i in range(N_UNITS_ // L):
        nstat_v[pl.ds(i * L, L)] = zf
    elbo_v[...] = zf

    def chunk_body(ci, elbo_c):
        base = wid * SPW + ci * CH
        pltpu.sync_copy(fT.at[:, pl.ds(base, CH)], fT_v)
        pltpu.sync_copy(lab_h.at[pl.ds(base, CH)], lab_v)
        pltpu.sync_copy(nb_h.at[pl.ds(base, CH)], nb_v)

        def group_body(g, elbo_acc):
            gb = g * L
            lab = lab_v[pl.ds(gb, L)]
            nbv = nb_v[pl.ds(gb, L)]
            iota = lax.iota(jnp.int32, L)
            sidx = base + gb + iota
            fds = [fT_v[d, pl.ds(gb, L)] for d in range(D_)]

            cands = []
            for j in range(3):
                cands.append(plsc.load_gather(
                    cl_v, [lab, jnp.full((L,), j, jnp.int32)]))
            for j in range(3):
                for k in range(2):
                    cands.append(plsc.load_gather(
                        usn_v, [cands[j], jnp.full((L,), k, jnp.int32)]))
            ne = plsc.load_gather(nexp_v, [nbv])
            ne = jnp.maximum(ne, 1)
            q0 = (sidx.astype(jnp.float32) / ne.astype(jnp.float32)
                  ).astype(jnp.int32)
            targ = sidx - q0 * ne
            targ = jnp.where(targ < 0, targ + ne, targ)
            targ = jnp.where(targ >= ne, targ - ne, targ)
            cands.append(plsc.load_gather(expl_v, [nbv, targ]))

            lls = []
            for j in range(10):
                cj = cands[j] * DP
                prods = []
                for d in range(D_):
                    md = plsc.load_gather(mu_v, [cj + d])
                    t = fds[d] - md
                    prods.append(t * t)
                while len(prods) > 1:
                    prods = [prods[i] + prods[i + 1]
                             for i in range(0, len(prods), 2)]
                lls.append(-0.5 * prods[0])

            neg_inf = jnp.full((L,), -jnp.inf, jnp.float32)
            chosen_pos, vals_sel, cand_sel = [], [], []
            for r in range(3):
                best, bp = neg_inf, jnp.zeros((L,), jnp.int32)
                bc = jnp.zeros((L,), jnp.int32)
                for j in range(10):
                    ok = lls[j] > best
                    for p in chosen_pos:
                        ok = ok & (p != j)
                    best = jnp.where(ok, lls[j], best)
                    bp = jnp.where(ok, jnp.full((L,), j, jnp.int32), bp)
                    bc = jnp.where(ok, cands[j], bc)
                chosen_pos.append(bp)
                vals_sel.append(best)
                cand_sel.append(bc)

            mx = jnp.maximum(vals_sel[0], jnp.maximum(vals_sel[1],
                                                      vals_sel[2]))
            es = [jnp.exp(v - mx) for v in vals_sel]
            ssum = (es[0] + es[1]) + es[2]
            qs = [e / ssum for e in es]

            for r in range(3):
                ntop_v[r, pl.ds(gb, L)] = cand_sel[r]
                q_v[r, pl.ds(gb, L)] = qs[r]

            contrib = (qs[0] * vals_sel[0] + qs[1] * vals_sel[1]
                       + qs[2] * vals_sel[2])

            for r in range(3):
                plsc.addupdate_scatter(nstat_v, [cand_sel[r]], qs[r])
            for r in range(3):
                c32 = cand_sel[r] * DP
                for d in range(D_):
                    plsc.addupdate_scatter(m_v, [c32 + d], qs[r] * fds[d])
            onesf = jnp.ones((L,), jnp.float32)
            for j in range(10):
                plsc.addupdate_scatter(
                    cnt_v, [cands[j] * N_NEIGHB_ + nbv], onesf)
            return elbo_acc + contrib
        elbo_c = plsc.parallel_loop(0, NG, carry=elbo_c)(group_body)

        pltpu.sync_copy(ntop_v, ntop_h.at[:, pl.ds(base, CH)])
        pltpu.sync_copy(q_v, q_h.at[:, pl.ds(base, CH)])
        return elbo_c
    elbo_fin = lax.fori_loop(0, NCH, chunk_body, jnp.zeros((L,), jnp.float32))
    elbo_v[...] = elbo_fin

    pltpu.sync_copy(nstat_v, nstat_h.at[wid])
    pltpu.sync_copy(m_v, m_h.at[wid])
    pltpu.sync_copy(cnt_v, cnt_h.at[wid])
    pltpu.sync_copy(elbo_v, elbo_h.at[wid])


def kernel(features, labels, neighborhood_ids, closest_neighbors,
           unit_search_neighbors, neighborhood_explore_units, unit_means):
    n_spikes, d = features.shape
    n_units = unit_means.shape[0]
    assert (n_spikes, d, n_units) == (N_SPIKES_, D_, N_UNITS_)

    fT = features.T
    mu_flat = jnp.pad(unit_means, ((0, 0), (0, DP - D_))).reshape(-1)

    mesh = plsc.VectorSubcoreMesh(core_axis_name="c", subcore_axis_name="s")
    f32, i32 = jnp.float32, jnp.int32
    out_type = (
        jax.ShapeDtypeStruct((3, N_SPIKES_), i32),
        jax.ShapeDtypeStruct((3, N_SPIKES_), f32),
        jax.ShapeDtypeStruct((NW, N_UNITS_), f32),
        jax.ShapeDtypeStruct((NW, N_UNITS_ * DP), f32),
        jax.ShapeDtypeStruct((NW, N_UNITS_ * N_NEIGHB_), f32),
        jax.ShapeDtypeStruct((NW, L), f32),
    )
    scratch_types = [
        pltpu.VMEM((N_UNITS_ * DP,), f32),
        pltpu.VMEM((N_UNITS_, 3), i32),
        pltpu.VMEM((N_UNITS_, 2), i32),
        pltpu.VMEM((N_NEIGHB_, N_UNITS_), i32),
        pltpu.VMEM((N_NEIGHB_,), i32),
        pltpu.VMEM((N_UNITS_,), f32),
        pltpu.VMEM((N_UNITS_ * DP,), f32),
        pltpu.VMEM((N_UNITS_ * N_NEIGHB_,), f32),
        pltpu.VMEM((D_, CH), f32),
        pltpu.VMEM((CH,), i32),
        pltpu.VMEM((CH,), i32),
        pltpu.VMEM((3, CH), i32),
        pltpu.VMEM((3, CH), f32),
        pltpu.VMEM((L,), f32),
        pltpu.SemaphoreType.DMA,
    ]
    run = pl.kernel(_sc_body, out_type=out_type, mesh=mesh,
                    scratch_types=scratch_types,
                    compiler_params=pltpu.CompilerParams(
                        use_tc_tiling_on_sc=False,
                        needs_layout_passes=False))
    ntop_t, q_t, nstat_p, m_p, cnt_p, elbo_p = run(
        fT, labels, neighborhood_ids, closest_neighbors,
        unit_search_neighbors, neighborhood_explore_units, mu_flat)

    new_top = ntop_t.T
    Q = q_t.T
    Nstat = nstat_p.sum(axis=0)
    m = m_p.sum(axis=0).reshape(N_UNITS_, DP)[:, :D_]
    counts = cnt_p.sum(axis=0).reshape(N_UNITS_, N_NEIGHB_)
    obs_elbo = elbo_p.sum()
    return new_top, Q, Nstat, m, counts, obs_elbo
